# baseline (speedup 1.0000x reference)
import functools

import jax
import jax.numpy as jnp
from jax import lax
from jax.experimental import pallas as pl
from jax.experimental.pallas import tpu as pltpu
from jax.experimental.pallas import tpu_sc as plsc

_G = 128
_CHUNK = 256
_K = 3
_L = 1


def _ceil_div(a, b):
    return (a + b - 1) // b


@functools.lru_cache(maxsize=None)
def _build(n_nodes, n_edges, hidden, n_nt, n_et):
    info = plsc.get_sparse_core_info()
    nc, ns = info.num_cores, info.num_subcores
    nw = nc * ns

    assert n_edges % nw == 0 and (n_edges // nw) % 8 == 0
    e_per_w = n_edges // nw
    e_full = e_per_w // _CHUNK
    e_tail = e_per_w % _CHUNK

    n_chunks = n_nodes // _CHUNK
    n_tail = n_nodes % _CHUNK
    n_lo = n_chunks // nw
    n_extra = n_chunks % nw
    n_hi = n_lo + (1 if n_extra else 0)
    n_ld = max(n_hi * _CHUNK, 8)
    assert n_lo >= _K and e_full >= _K

    mesh = plsc.VectorSubcoreMesh(core_axis_name="c", subcore_axis_name="s")

    scratch = (
        [pltpu.VMEM((e_per_w,), jnp.int32),
         pltpu.VMEM((n_ld,), jnp.int32),
         pltpu.VMEM_SHARED((n_nt, hidden), jnp.float32),
         pltpu.VMEM_SHARED((n_et, hidden), jnp.float32)]
        + [pltpu.VMEM((_CHUNK, hidden), jnp.float32) for _ in range(_K)]
        + [pltpu.SemaphoreType.DMA for _ in range(2 * _K)]
    )

    @functools.partial(
        pl.kernel,
        mesh=mesh,
        out_type=jax.ShapeDtypeStruct((n_nodes + n_edges, hidden), jnp.float32),
        scratch_types=scratch,
    )
    def k(node_ids, edge_ids, ntab, etab, out,
          e_ids, n_ids, ntab_sp, etab_sp, *bufs_sems):
        bufs = bufs_sems[:_K]
        gsem = bufs_sems[_K:2 * _K]
        ssem = bufs_sems[2 * _K:]
        wid = lax.axis_index("s") * nc + lax.axis_index("c")

        @pl.when(lax.axis_index("s") == 0)
        def _():
            pltpu.sync_copy(ntab, ntab_sp)
            pltpu.sync_copy(etab, etab_sp)

        plsc.subcore_barrier()

        def ring(ids_v, id_shift, tab_sp, out_base, my_n, n_bound):
            def start_gather(p, pb):
                for g in range(_CHUNK // _G):
                    off = id_shift + p * _CHUNK + g * _G
                    src = tab_sp.at[ids_v.at[pl.ds(off, _G)]]
                    dst = bufs[pb].at[pl.ds(g * _G, _G)]
                    pltpu.make_async_copy(src, dst, gsem[pb]).start()

            def wait_gather(b):
                for g in range(_CHUNK // _G):
                    pltpu.make_async_copy(
                        tab_sp.at[ids_v.at[pl.ds(0, _G)]],
                        bufs[b].at[pl.ds(g * _G, _G)], gsem[b]).wait()

            def start_scatter(i, b):
                dst = out.at[pl.ds(out_base + i * _CHUNK, _CHUNK)]
                pltpu.make_async_copy(bufs[b], dst, ssem[b]).start()

            def wait_scatter(b):
                pltpu.make_async_copy(
                    bufs[b], out.at[pl.ds(0, _CHUNK)], ssem[b]).wait()

            for b0 in range(_L):
                start_gather(b0, b0)

            def body(m, carry):
                for b in range(_K):
                    i = m * _K + b
                    p = i + _L
                    pb = (b + _L) % _K

                    @pl.when(jnp.logical_and(p < my_n, p >= _K))
                    def _(pb=pb):
                        wait_scatter(pb)

                    @pl.when(p < my_n)
                    def _(p=p, pb=pb):
                        start_gather(p, pb)

                    @pl.when(i < my_n)
                    def _(i=i, b=b):
                        wait_gather(b)
                        start_scatter(i, b)

                return carry

            lax.fori_loop(0, _ceil_div(n_bound, _K), body, 0)
            for b in range(_K):
                wait_scatter(b)

        e_base = wid * e_per_w
        pltpu.sync_copy(edge_ids.at[pl.ds(e_base, e_per_w)], e_ids)
        ring(e_ids, 0, etab_sp, n_nodes + e_base, e_full, e_full)

        if e_tail:
            off = e_full * _CHUNK
            done = 0
            while done < e_tail:
                step = min(_G, e_tail - done)
                src = etab_sp.at[e_ids.at[pl.ds(off + done, step)]]
                pltpu.async_copy(src, bufs[0].at[pl.ds(done, step)],
                                 gsem[0]).wait()
                done += step
            pltpu.sync_copy(bufs[0].at[pl.ds(0, e_tail)],
                            out.at[pl.ds(n_nodes + e_base + off, e_tail)])

        nbase_chunk = wid * n_lo + jnp.minimum(wid, n_extra)
        nbase_ids = nbase_chunk * _CHUNK
        ld_off = jnp.minimum(nbase_ids, n_nodes - n_ld)
        pltpu.sync_copy(node_ids.at[pl.ds(ld_off, n_ld)],
                        n_ids.at[pl.ds(0, n_ld)])
        my_nn = jnp.where(wid < n_extra, n_hi, n_lo)
        ring(n_ids, nbase_ids - ld_off, ntab_sp, nbase_ids, my_nn, n_hi)

        if n_tail:
            @pl.when(wid == nw - 1)
            def _():
                off = n_chunks * _CHUNK
                pltpu.sync_copy(node_ids.at[pl.ds(off, n_tail)],
                                n_ids.at[pl.ds(0, n_tail)])
                src = ntab_sp.at[n_ids.at[pl.ds(0, n_tail)]]
                pltpu.async_copy(src, bufs[0].at[pl.ds(0, n_tail)],
                                 gsem[0]).wait()
                pltpu.sync_copy(bufs[0].at[pl.ds(0, n_tail)],
                                out.at[pl.ds(off, n_tail)])

    return k


def kernel(node_type_ids, edge_type_ids, node_type_table, edge_type_table):
    n_nodes = node_type_ids.shape[0]
    n_edges = edge_type_ids.shape[0]
    hidden = node_type_table.shape[1]
    k = _build(n_nodes, n_edges, hidden,
               node_type_table.shape[0], edge_type_table.shape[0])
    return k(node_type_ids.astype(jnp.int32), edge_type_ids.astype(jnp.int32),
             node_type_table, edge_type_table)


# speedup vs baseline: 1.0476x; 1.0476x over previous
import functools

import jax
import jax.numpy as jnp
from jax import lax
from jax.experimental import pallas as pl
from jax.experimental.pallas import tpu as pltpu
from jax.experimental.pallas import tpu_sc as plsc

_CHUNK = 128
_K = 6
_L = 4


def _ceil_div(a, b):
    return (a + b - 1) // b


@functools.lru_cache(maxsize=None)
def _build(n_nodes, n_edges, hidden, n_nt, n_et):
    info = plsc.get_sparse_core_info()
    nc, ns = info.num_cores, info.num_subcores
    nw = nc * ns

    assert n_edges % nw == 0 and (n_edges // nw) % 8 == 0
    e_per_w = n_edges // nw
    e_full = e_per_w // _CHUNK
    e_tail = e_per_w % _CHUNK

    n_chunks = n_nodes // _CHUNK
    n_tail = n_nodes % _CHUNK
    n_lo = n_chunks // nw
    n_extra = n_chunks % nw
    n_hi = n_lo + (1 if n_extra else 0)
    n_ld = max(n_hi * _CHUNK, 8)
    assert n_lo >= _K and e_full >= _K

    mesh = plsc.VectorSubcoreMesh(core_axis_name="c", subcore_axis_name="s")

    scratch = (
        [pltpu.VMEM((e_per_w,), jnp.int32),
         pltpu.VMEM((n_ld,), jnp.int32),
         pltpu.VMEM_SHARED((n_nt, hidden), jnp.float32),
         pltpu.VMEM_SHARED((n_et, hidden), jnp.float32)]
        + [pltpu.VMEM((_CHUNK, hidden), jnp.float32) for _ in range(_K)]
        + [pltpu.SemaphoreType.DMA for _ in range(2 * _K)]
    )

    @functools.partial(
        pl.kernel,
        mesh=mesh,
        out_type=jax.ShapeDtypeStruct((n_nodes + n_edges, hidden), jnp.float32),
        scratch_types=scratch,
    )
    def k(node_ids, edge_ids, ntab, etab, out,
          e_ids, n_ids, ntab_sp, etab_sp, *bufs_sems):
        bufs = bufs_sems[:_K]
        gsem = bufs_sems[_K:2 * _K]
        ssem = bufs_sems[2 * _K:]
        wid = lax.axis_index("s") * nc + lax.axis_index("c")

        @pl.when(lax.axis_index("s") == 0)
        def _():
            pltpu.sync_copy(ntab, ntab_sp)
            pltpu.sync_copy(etab, etab_sp)

        plsc.subcore_barrier()

        def ring(ids_v, id_shift, tab_sp, out_base, my_n, n_bound):
            def start_gather(p, pb):
                src = tab_sp.at[ids_v.at[pl.ds(id_shift + p * _CHUNK, _CHUNK)]]
                pltpu.make_async_copy(src, bufs[pb], gsem[pb]).start()

            def wait_gather(b):
                pltpu.make_async_copy(
                    tab_sp.at[ids_v.at[pl.ds(0, _CHUNK)]],
                    bufs[b], gsem[b]).wait()

            def start_scatter(i, b):
                dst = out.at[pl.ds(out_base + i * _CHUNK, _CHUNK)]
                pltpu.make_async_copy(bufs[b], dst, ssem[b]).start()

            def wait_scatter(b):
                pltpu.make_async_copy(
                    bufs[b], out.at[pl.ds(0, _CHUNK)], ssem[b]).wait()

            for b0 in range(_L):
                start_gather(b0, b0)

            def body(m, carry):
                for b in range(_K):
                    i = m * _K + b
                    p = i + _L
                    pb = (b + _L) % _K

                    @pl.when(jnp.logical_and(p < my_n, p >= _K))
                    def _(pb=pb):
                        wait_scatter(pb)

                    @pl.when(p < my_n)
                    def _(p=p, pb=pb):
                        start_gather(p, pb)

                    @pl.when(i < my_n)
                    def _(i=i, b=b):
                        wait_gather(b)
                        start_scatter(i, b)

                return carry

            lax.fori_loop(0, _ceil_div(n_bound, _K), body, 0)
            for b in range(_K):
                wait_scatter(b)

        e_base = wid * e_per_w
        pltpu.sync_copy(edge_ids.at[pl.ds(e_base, e_per_w)], e_ids)
        ring(e_ids, 0, etab_sp, n_nodes + e_base, e_full, e_full)

        if e_tail:
            off = e_full * _CHUNK
            src = etab_sp.at[e_ids.at[pl.ds(off, e_tail)]]
            pltpu.async_copy(src, bufs[0].at[pl.ds(0, e_tail)], gsem[0]).wait()
            pltpu.sync_copy(bufs[0].at[pl.ds(0, e_tail)],
                            out.at[pl.ds(n_nodes + e_base + off, e_tail)])

        nbase_chunk = wid * n_lo + jnp.minimum(wid, n_extra)
        nbase_ids = nbase_chunk * _CHUNK
        ld_off = jnp.minimum(nbase_ids, n_nodes - n_ld)
        pltpu.sync_copy(node_ids.at[pl.ds(ld_off, n_ld)],
                        n_ids.at[pl.ds(0, n_ld)])
        my_nn = jnp.where(wid < n_extra, n_hi, n_lo)
        ring(n_ids, nbase_ids - ld_off, ntab_sp, nbase_ids, my_nn, n_hi)

        if n_tail:
            @pl.when(wid == nw - 1)
            def _():
                off = n_chunks * _CHUNK
                pltpu.sync_copy(node_ids.at[pl.ds(off, n_tail)],
                                n_ids.at[pl.ds(0, n_tail)])
                src = ntab_sp.at[n_ids.at[pl.ds(0, n_tail)]]
                pltpu.async_copy(src, bufs[0].at[pl.ds(0, n_tail)],
                                 gsem[0]).wait()
                pltpu.sync_copy(bufs[0].at[pl.ds(0, n_tail)],
                                out.at[pl.ds(off, n_tail)])

    return k


def kernel(node_type_ids, edge_type_ids, node_type_table, edge_type_table):
    n_nodes = node_type_ids.shape[0]
    n_edges = edge_type_ids.shape[0]
    hidden = node_type_table.shape[1]
    k = _build(n_nodes, n_edges, hidden,
               node_type_table.shape[0], edge_type_table.shape[0])
    return k(node_type_ids.astype(jnp.int32), edge_type_ids.astype(jnp.int32),
             node_type_table, edge_type_table)


# speedup vs baseline: 1.0488x; 1.0012x over previous
import functools

import jax
import jax.numpy as jnp
from jax import lax
from jax.experimental import pallas as pl
from jax.experimental.pallas import tpu as pltpu
from jax.experimental.pallas import tpu_sc as plsc

_CHUNK = 128
_K = 6
_L = 4


def _ceil_div(a, b):
    return (a + b - 1) // b


@functools.lru_cache(maxsize=None)
def _build(n_nodes, n_edges, hidden, n_nt, n_et):
    info = plsc.get_sparse_core_info()
    nc, ns = info.num_cores, info.num_subcores
    nw = nc * ns

    assert n_edges % nw == 0 and (n_edges // nw) % 8 == 0
    e_per_w = n_edges // nw
    e_full = e_per_w // _CHUNK
    e_tail = e_per_w % _CHUNK

    n_chunks = n_nodes // _CHUNK
    n_tail = n_nodes % _CHUNK
    n_lo = n_chunks // nw
    n_extra = n_chunks % nw
    n_hi = n_lo + (1 if n_extra else 0)
    n_ld = max(n_hi * _CHUNK, 8)
    assert n_lo >= _K and e_full >= _K

    mesh = plsc.VectorSubcoreMesh(core_axis_name="c", subcore_axis_name="s")

    scratch = (
        [pltpu.VMEM((e_per_w,), jnp.int32),
         pltpu.VMEM((n_ld,), jnp.int32),
         pltpu.VMEM_SHARED((n_nt, hidden), jnp.float32),
         pltpu.VMEM_SHARED((n_et, hidden), jnp.float32)]
        + [pltpu.VMEM((_CHUNK, hidden), jnp.float32) for _ in range(_K)]
        + [pltpu.SemaphoreType.DMA for _ in range(2 * _K)]
    )

    @functools.partial(
        pl.kernel,
        mesh=mesh,
        out_type=jax.ShapeDtypeStruct((n_nodes + n_edges, hidden), jnp.float32),
        scratch_types=scratch,
    )
    def k(node_ids, edge_ids, ntab, etab, out,
          e_ids, n_ids, ntab_sp, etab_sp, *bufs_sems):
        bufs = bufs_sems[:_K]
        gsem = bufs_sems[_K:2 * _K]
        ssem = bufs_sems[2 * _K:]
        wid = lax.axis_index("s") * nc + lax.axis_index("c")

        @pl.when(lax.axis_index("s") == 0)
        def _():
            pltpu.sync_copy(ntab, ntab_sp)
            pltpu.sync_copy(etab, etab_sp)

        plsc.subcore_barrier()

        e_base = wid * e_per_w
        pltpu.sync_copy(edge_ids.at[pl.ds(e_base, e_per_w)],
                        e_ids.at[pl.ds(0, e_per_w)])
        nbase_chunk = wid * n_lo + jnp.minimum(wid, n_extra)
        nbase_ids = nbase_chunk * _CHUNK
        ld_off = jnp.minimum(nbase_ids, n_nodes - n_ld)
        pltpu.sync_copy(node_ids.at[pl.ds(ld_off, n_ld)],
                        n_ids.at[pl.ds(0, n_ld)])
        n_shift = nbase_ids - ld_off
        my_nn = jnp.where(wid < n_extra, n_hi, n_lo)

        my_n = e_full + my_nn
        n_bound = e_full + n_hi

        def start_gather(p, pb):
            @pl.when(p < e_full)
            def _():
                src = etab_sp.at[e_ids.at[pl.ds(p * _CHUNK, _CHUNK)]]
                pltpu.make_async_copy(src, bufs[pb], gsem[pb]).start()

            @pl.when(p >= e_full)
            def _():
                off = n_shift + (p - e_full) * _CHUNK
                src = ntab_sp.at[n_ids.at[pl.ds(off, _CHUNK)]]
                pltpu.make_async_copy(src, bufs[pb], gsem[pb]).start()

        def wait_gather(b):
            pltpu.make_async_copy(
                etab_sp.at[e_ids.at[pl.ds(0, _CHUNK)]],
                bufs[b], gsem[b]).wait()

        def start_scatter(i, b):
            row = jnp.where(i < e_full,
                            n_nodes + e_base + i * _CHUNK,
                            (nbase_chunk + i - e_full) * _CHUNK)
            dst = out.at[pl.ds(row, _CHUNK)]
            pltpu.make_async_copy(bufs[b], dst, ssem[b]).start()

        def wait_scatter(b):
            pltpu.make_async_copy(
                bufs[b], out.at[pl.ds(0, _CHUNK)], ssem[b]).wait()

        for b0 in range(_L):
            start_gather(b0, b0)

        def body(m, carry):
            for b in range(_K):
                i = m * _K + b
                p = i + _L
                pb = (b + _L) % _K

                @pl.when(jnp.logical_and(p < my_n, p >= _K))
                def _(pb=pb):
                    wait_scatter(pb)

                @pl.when(p < my_n)
                def _(p=p, pb=pb):
                    start_gather(p, pb)

                @pl.when(i < my_n)
                def _(i=i, b=b):
                    wait_gather(b)
                    start_scatter(i, b)

            return carry

        lax.fori_loop(0, _ceil_div(n_bound, _K), body, 0)
        for b in range(_K):
            wait_scatter(b)

        if e_tail:
            off = e_full * _CHUNK
            src = etab_sp.at[e_ids.at[pl.ds(off, e_tail)]]
            pltpu.async_copy(src, bufs[0].at[pl.ds(0, e_tail)], gsem[0]).wait()
            pltpu.sync_copy(bufs[0].at[pl.ds(0, e_tail)],
                            out.at[pl.ds(n_nodes + e_base + off, e_tail)])

        if n_tail:
            @pl.when(wid == nw - 1)
            def _():
                off = n_chunks * _CHUNK
                pltpu.sync_copy(node_ids.at[pl.ds(off, n_tail)],
                                n_ids.at[pl.ds(0, n_tail)])
                src = ntab_sp.at[n_ids.at[pl.ds(0, n_tail)]]
                pltpu.async_copy(src, bufs[0].at[pl.ds(0, n_tail)],
                                 gsem[0]).wait()
                pltpu.sync_copy(bufs[0].at[pl.ds(0, n_tail)],
                                out.at[pl.ds(off, n_tail)])

    return k


def kernel(node_type_ids, edge_type_ids, node_type_table, edge_type_table):
    n_nodes = node_type_ids.shape[0]
    n_edges = edge_type_ids.shape[0]
    hidden = node_type_table.shape[1]
    k = _build(n_nodes, n_edges, hidden,
               node_type_table.shape[0], edge_type_table.shape[0])
    return k(node_type_ids.astype(jnp.int32), edge_type_ids.astype(jnp.int32),
             node_type_table, edge_type_table)
